# baseline (speedup 1.0000x reference)
import functools

import jax
import jax.numpy as jnp
from jax import lax
from jax.experimental import pallas as pl
from jax.experimental.pallas import tpu as pltpu
from jax.experimental.pallas import tpu_sc as plsc

B, N, P, LRF, M = 4, 8192, 1024, 4, 2048

NC, NS, L = 2, 16, 16
NW = NC * NS
PB = L
NCB = P // PB
UNITS = B * NCB
UPW = UNITS // NW
NH = 2
NHALF = N // NH
NTH = NHALF // 8
MC = 128
NMC = M // MC

_mesh = plsc.VectorSubcoreMesh(core_axis_name="c", subcore_axis_name="s")


@functools.partial(
    pl.kernel,
    out_type=jax.ShapeDtypeStruct((B, M, P), jnp.float32),
    mesh=_mesh,
    scratch_types=[
        pltpu.VMEM((NTH, 8, PB), jnp.float32),
        pltpu.VMEM((M, PB), jnp.float32),
        pltpu.VMEM((MC, LRF, PB), jnp.int32),
        pltpu.VMEM((MC, LRF, PB), jnp.int32),
        pltpu.SemaphoreType.DMA,
        pltpu.SemaphoreType.DMA,
    ],
    compiler_params=pltpu.CompilerParams(
        use_tc_tiling_on_sc=False, needs_layout_passes=False
    ),
)
def _mc_max_pool(x_hbm, lrf_hbm, out_hbm, xbuf, acc, ibuf0, ibuf1, sem0, sem1):
    wid = lax.axis_index("s") * NC + lax.axis_index("c")
    neg_inf = jnp.full((L,), -jnp.inf, dtype=jnp.float32)
    lane = jnp.arange(L, dtype=jnp.int32)
    zero = jnp.zeros((L,), dtype=jnp.int32)

    def unit_body(u, carry):
        unit = wid * UPW + u
        b = unit // NCB
        cb = unit % NCB
        pt = cb // 8
        cl0 = (cb % 8) * PB

        def lrf_chunk(mc):
            return lrf_hbm.at[pl.ds(mc * MC, MC), pt, :, pl.ds(cl0, PB)]

        for h in range(NH):
            pltpu.sync_copy(
                x_hbm.at[b, pl.ds(h * NTH, NTH), pt, :, pl.ds(cl0, PB)], xbuf
            )
            pltpu.async_copy(lrf_chunk(0), ibuf0, sem0)

            def process(ibuf, mc, h=h):
                def m_body(mi):
                    vmax = None
                    for k in range(LRF):
                        idx = ibuf[mi, k]
                        flat = jnp.left_shift(idx, 4) + lane
                        if h == 0:
                            msk = flat < NHALF * L
                            jc = jnp.minimum(flat, NHALF * L - 1)
                        else:
                            jloc = flat - NHALF * L
                            msk = jloc >= 0
                            jc = jnp.maximum(jloc, 0)
                        v = plsc.load_gather(xbuf, [zero, zero, jc])
                        v = jnp.where(msk, v, neg_inf)
                        vmax = v if vmax is None else jnp.maximum(vmax, v)
                    row = mc * MC + mi
                    if h == 0:
                        acc[row] = vmax
                    else:
                        acc[row] = jnp.maximum(acc[row], vmax)

                plsc.parallel_loop(0, MC, unroll=2)(m_body)

            def pair_body(mcp, _, h=h):
                mc0 = 2 * mcp
                pltpu.make_async_copy(lrf_chunk(mc0), ibuf0, sem0).wait()
                pltpu.async_copy(lrf_chunk(mc0 + 1), ibuf1, sem1)
                process(ibuf0, mc0)
                pltpu.make_async_copy(lrf_chunk(mc0 + 1), ibuf1, sem1).wait()

                @pl.when(mcp < NMC // 2 - 1)
                def _prefetch():
                    pltpu.async_copy(lrf_chunk(mc0 + 2), ibuf0, sem0)

                process(ibuf1, mc0 + 1)
                return _

            lax.fori_loop(0, NMC // 2, pair_body, None)

        pltpu.sync_copy(acc, out_hbm.at[b, :, pl.ds(cb * PB, PB)])
        return carry

    lax.fori_loop(0, UPW, unit_body, None)


def kernel(x, lrf_idx):
    x5 = jnp.transpose(x.reshape(B, N // 8, 8, P // 128, 128), (0, 1, 3, 2, 4))
    lrf4 = jnp.transpose(lrf_idx.reshape(M, 8, 128, LRF), (0, 1, 3, 2))
    return _mc_max_pool(x5, lrf4)


# speedup vs baseline: 1.1708x; 1.1708x over previous
import functools

import jax
import jax.numpy as jnp
from jax import lax
from jax.experimental import pallas as pl
from jax.experimental.pallas import tpu as pltpu
from jax.experimental.pallas import tpu_sc as plsc

B, N, P, LRF, M = 4, 8192, 1024, 4, 2048

NC, NS, L = 2, 16, 16
NW = NC * NS
PB = L
NCB = P // PB
UNITS = B * NCB
UPW = UNITS // NW
NH = 2
NHALF = N // NH
NTH = NHALF // 8
MC = 128
NMC = M // MC

_mesh = plsc.VectorSubcoreMesh(core_axis_name="c", subcore_axis_name="s")


@functools.partial(
    pl.kernel,
    out_type=jax.ShapeDtypeStruct((B, M, P), jnp.float32),
    mesh=_mesh,
    scratch_types=[
        pltpu.VMEM((NTH, 8, PB), jnp.float32),
        pltpu.VMEM((M, PB), jnp.float32),
        pltpu.VMEM((MC, LRF, PB), jnp.int32),
        pltpu.VMEM((MC, LRF, PB), jnp.int32),
        pltpu.SemaphoreType.DMA,
        pltpu.SemaphoreType.DMA,
    ],
    compiler_params=pltpu.CompilerParams(
        use_tc_tiling_on_sc=False, needs_layout_passes=False
    ),
)
def _mc_max_pool(x_hbm, lrf_hbm, out_hbm, xbuf, acc, ibuf0, ibuf1, sem0, sem1):
    wid = lax.axis_index("s") * NC + lax.axis_index("c")
    neg_inf = jnp.full((L,), -jnp.inf, dtype=jnp.float32)
    lane = jnp.arange(L, dtype=jnp.int32)
    zero = jnp.zeros((L,), dtype=jnp.int32)

    def unit_body(u, carry):
        unit = wid * UPW + u
        b = unit // NCB
        cb = unit % NCB
        pt = cb // 8
        cl0 = (cb % 8) * PB

        def lrf_chunk(mc):
            return lrf_hbm.at[pl.ds(mc * MC, MC), pt, :, pl.ds(cl0, PB)]

        for h in range(NH):
            pltpu.sync_copy(
                x_hbm.at[b, pl.ds(h * NTH, NTH), pt, :, pl.ds(cl0, PB)], xbuf
            )
            pltpu.async_copy(lrf_chunk(0), ibuf0, sem0)

            def process(ibuf, mc, h=h):
                def m_body(mi):
                    vmax = None
                    for k in range(LRF):
                        idx = ibuf[mi, k]
                        if h == 0:
                            msk = idx < NHALF
                            jc = jnp.minimum(idx, NHALF - 1)
                        else:
                            jloc = idx - NHALF
                            msk = jloc >= 0
                            jc = jnp.maximum(jloc, 0)
                        v = plsc.load_gather(
                            xbuf,
                            [
                                jnp.right_shift(jc, 3),
                                jnp.bitwise_and(jc, 7),
                                lane,
                            ],
                        )
                        v = jnp.where(msk, v, neg_inf)
                        vmax = v if vmax is None else jnp.maximum(vmax, v)
                    row = mc * MC + mi
                    if h == 0:
                        acc[row] = vmax
                    else:
                        acc[row] = jnp.maximum(acc[row], vmax)

                plsc.parallel_loop(0, MC, unroll=2)(m_body)

            def pair_body(mcp, _, h=h):
                mc0 = 2 * mcp
                pltpu.make_async_copy(lrf_chunk(mc0), ibuf0, sem0).wait()
                pltpu.async_copy(lrf_chunk(mc0 + 1), ibuf1, sem1)
                process(ibuf0, mc0)
                pltpu.make_async_copy(lrf_chunk(mc0 + 1), ibuf1, sem1).wait()

                @pl.when(mcp < NMC // 2 - 1)
                def _prefetch():
                    pltpu.async_copy(lrf_chunk(mc0 + 2), ibuf0, sem0)

                process(ibuf1, mc0 + 1)
                return _

            lax.fori_loop(0, NMC // 2, pair_body, None)

        pltpu.sync_copy(acc, out_hbm.at[b, :, pl.ds(cb * PB, PB)])
        return carry

    lax.fori_loop(0, UPW, unit_body, None)


def kernel(x, lrf_idx):
    x5 = jnp.transpose(x.reshape(B, N // 8, 8, P // 128, 128), (0, 1, 3, 2, 4))
    lrf4 = jnp.transpose(lrf_idx.reshape(M, 8, 128, LRF), (0, 1, 3, 2))
    return _mc_max_pool(x5, lrf4)


# speedup vs baseline: 1.1742x; 1.0029x over previous
import functools

import jax
import jax.numpy as jnp
from jax import lax
from jax.experimental import pallas as pl
from jax.experimental.pallas import tpu as pltpu
from jax.experimental.pallas import tpu_sc as plsc

B, N, P, LRF, M = 4, 8192, 1024, 4, 2048

NC, NS, L = 2, 16, 16
NW = NC * NS
PB = L
NCB = P // PB
UNITS = B * NCB
UPW = UNITS // NW
NH = 2
NHALF = N // NH
NTH = NHALF // 8
MC = 128
NMC = M // MC

_mesh = plsc.VectorSubcoreMesh(core_axis_name="c", subcore_axis_name="s")


@functools.partial(
    pl.kernel,
    out_type=jax.ShapeDtypeStruct((B, M, P), jnp.float32),
    mesh=_mesh,
    scratch_types=[
        pltpu.VMEM((NTH, 8, PB), jnp.float32),
        pltpu.VMEM((M, PB), jnp.float32),
        pltpu.VMEM((MC, LRF, PB), jnp.int32),
        pltpu.VMEM((MC, LRF, PB), jnp.int32),
        pltpu.SemaphoreType.DMA,
        pltpu.SemaphoreType.DMA,
    ],
    compiler_params=pltpu.CompilerParams(
        use_tc_tiling_on_sc=False, needs_layout_passes=False
    ),
)
def _mc_max_pool(x_hbm, lrf_hbm, out_hbm, xbuf, acc, ibuf0, ibuf1, sem0, sem1):
    wid = lax.axis_index("s") * NC + lax.axis_index("c")
    neg_inf = jnp.full((L,), -jnp.inf, dtype=jnp.float32)
    lane = jnp.arange(L, dtype=jnp.int32)

    def unit_body(u, carry):
        unit = wid * UPW + u
        b = unit // NCB
        cb = unit % NCB
        pt = cb // 8
        cl0 = (cb % 8) * PB

        def lrf_chunk(mc):
            return lrf_hbm.at[pl.ds(mc * MC, MC), pt, :, pl.ds(cl0, PB)]

        for h in range(NH):
            pltpu.sync_copy(
                x_hbm.at[b, pl.ds(h * NTH, NTH), pt, :, pl.ds(cl0, PB)], xbuf
            )
            pltpu.async_copy(lrf_chunk(0), ibuf0, sem0)

            def process(ibuf, mc, h=h):
                def m_body(mi):
                    vmax = None
                    for k in range(LRF):
                        idx = ibuf[mi, k]
                        if h == 0:
                            msk = idx < NHALF
                            jc = jnp.minimum(idx, NHALF - 1)
                        else:
                            jloc = idx - NHALF
                            msk = jloc >= 0
                            jc = jnp.maximum(jloc, 0)
                        v = plsc.load_gather(
                            xbuf,
                            [
                                jnp.right_shift(jc, 3),
                                jnp.bitwise_and(jc, 7),
                                lane,
                            ],
                        )
                        v = jnp.where(msk, v, neg_inf)
                        vmax = v if vmax is None else jnp.maximum(vmax, v)
                    row = mc * MC + mi
                    if h == 0:
                        acc[row] = vmax
                    else:
                        acc[row] = jnp.maximum(acc[row], vmax)

                plsc.parallel_loop(0, MC, unroll=2)(m_body)

            def pair_body(mcp, _, h=h):
                mc0 = 2 * mcp
                pltpu.make_async_copy(lrf_chunk(mc0), ibuf0, sem0).wait()
                pltpu.async_copy(lrf_chunk(mc0 + 1), ibuf1, sem1)
                process(ibuf0, mc0)
                pltpu.make_async_copy(lrf_chunk(mc0 + 1), ibuf1, sem1).wait()

                @pl.when(mcp < NMC // 2 - 1)
                def _prefetch():
                    pltpu.async_copy(lrf_chunk(mc0 + 2), ibuf0, sem0)

                process(ibuf1, mc0 + 1)
                return _

            lax.fori_loop(0, NMC // 2, pair_body, None)

        pltpu.sync_copy(acc, out_hbm.at[b, :, pl.ds(cb * PB, PB)])
        return carry

    lax.fori_loop(0, UPW, unit_body, None)


def kernel(x, lrf_idx):
    x5 = jnp.transpose(x.reshape(B, N // 8, 8, P // 128, 128), (0, 1, 3, 2, 4))
    lrf4 = jnp.transpose(lrf_idx.reshape(M, 8, 128, LRF), (0, 1, 3, 2))
    return _mc_max_pool(x5, lrf4)
